# Initial kernel scaffold; baseline (speedup 1.0000x reference)
import jax
import jax.numpy as jnp
from jax.experimental import pallas as pl


def kernel(input, points, support_points, indices, W, bbias):
    raise NotImplementedError("write your pallas kernel here")


# speedup vs baseline: 1312.6350x; 1312.6350x over previous
import functools

import jax
import jax.numpy as jnp
from jax import lax
from jax.experimental import pallas as pl
from jax.experimental.pallas import tpu as pltpu
from jax.experimental.pallas import tpu_sc as plsc

B, C_IN, N = 8, 64, 16384
NS, K = 4096, 16
C_OUT = 128
L = 16

NC, NSUB = 2, 16
NW = NC * NSUB
CH = (B * NS) // NW
WPB = NW // B
SUBC = 256
NSUBCH = CH // SUBC
GRP = 8
NGRP = SUBC // GRP


def _z_body(x_ref, p_ref, w1_ref, w2_ref, o_ref):
    z = lax.dot_general(x_ref[0], w1_ref[...], (((0,), (0,)), ((), ())),
                        preferred_element_type=jnp.float32)
    z = z + lax.dot_general(p_ref[0], w2_ref[...], (((0,), (0,)), ((), ())),
                            preferred_element_type=jnp.float32)
    o_ref[0] = z.astype(jnp.bfloat16)


def _s_body(sp_ref, w2_ref, b_ref, o_ref):
    s = lax.dot_general(sp_ref[0], w2_ref[...], (((0,), (0,)), ((), ())),
                        preferred_element_type=jnp.float32)
    o_ref[0] = (s - b_ref[...]).astype(jnp.bfloat16)


def _sc_body(z_hbm, idx_hbm, s_hbm, out_hbm,
             idx_v, s_v, buf_a, buf_b, out_t, sem_a, sem_b):
    cid = lax.axis_index("c")
    sid = lax.axis_index("s")
    wid = sid * NC + cid
    b = wid // WPB
    nbase = (wid % WPB) * CH

    def start_gather(g, buf, sem):
        off = pl.multiple_of(g * (GRP * K), GRP * K)
        pltpu.make_async_copy(
            z_hbm.at[idx_v.at[pl.ds(off, GRP * K)]], buf, sem).start()

    def wait_gather(g, buf, sem):
        off = pl.multiple_of(g * (GRP * K), GRP * K)
        pltpu.make_async_copy(
            z_hbm.at[idx_v.at[pl.ds(off, GRP * K)]], buf, sem).wait()

    def compute_group(g, buf):
        pp0 = g * GRP
        iota2 = lax.iota(jnp.int32, L) * 2
        himask = jnp.full((L,), -65536, jnp.int32)
        for p in range(GRP):
            pp = pp0 + p
            cols = jnp.full((L,), pp, jnp.int32)
            for q in range(C_OUT // (2 * L)):
                sl = pl.ds(2 * L * q, 2 * L)
                acc = buf[K * p, sl]
                for r in range(1, K):
                    acc = jnp.maximum(acc, buf[K * p + r, sl])
                mi = plsc.bitcast(acc, jnp.int32)
                lo = plsc.bitcast(mi << 16, jnp.float32)
                hi = plsc.bitcast(mi & himask, jnp.float32)
                svi = plsc.bitcast(s_v[pp, sl], jnp.int32)
                s_lo = plsc.bitcast(svi << 16, jnp.float32)
                s_hi = plsc.bitcast(svi & himask, jnp.float32)
                rows_lo = iota2 + (2 * L * q)
                plsc.store_scatter(out_t, [rows_lo, cols], lo - s_lo)
                plsc.store_scatter(out_t, [rows_lo + 1, cols], hi - s_hi)

    def sub_body(s_i, carry):
        base = pl.multiple_of(wid * CH + s_i * SUBC, SUBC)
        pltpu.sync_copy(idx_hbm.at[pl.ds(base * K, SUBC * K)], idx_v)
        pltpu.sync_copy(s_hbm.at[pl.ds(base, SUBC)], s_v)

        boff = b * N

        def shift_body(i, c):
            sl = pl.ds(pl.multiple_of(i * L, L), L)
            idx_v[sl] = idx_v[sl] + boff
            return c
        lax.fori_loop(0, (SUBC * K) // L, shift_body, 0)

        start_gather(0, buf_a, sem_a)

        def grp_body(i, c):
            g0 = 2 * i
            g1 = g0 + 1
            start_gather(g1, buf_b, sem_b)
            wait_gather(g0, buf_a, sem_a)
            compute_group(g0, buf_a)

            @pl.when(g1 + 1 < NGRP)
            def _():
                start_gather(g1 + 1, buf_a, sem_a)

            wait_gather(g1, buf_b, sem_b)
            compute_group(g1, buf_b)
            return c
        lax.fori_loop(0, NGRP // 2, grp_body, 0)

        n0 = pl.multiple_of(nbase + s_i * SUBC, SUBC)
        pltpu.sync_copy(out_t, out_hbm.at[b, :, pl.ds(n0, SUBC)])
        return carry
    lax.fori_loop(0, NSUBCH, sub_body, 0)


def kernel(input, points, support_points, indices, W, bbias):
    w1 = W[:C_IN]
    w2 = W[C_IN:]
    TN = 2048

    z = pl.pallas_call(
        _z_body,
        grid=(B, N // TN),
        in_specs=[
            pl.BlockSpec((1, C_IN, TN), lambda b, t: (b, 0, t)),
            pl.BlockSpec((1, 3, TN), lambda b, t: (b, 0, t)),
            pl.BlockSpec((C_IN, C_OUT), lambda b, t: (0, 0)),
            pl.BlockSpec((3, C_OUT), lambda b, t: (0, 0)),
        ],
        out_specs=pl.BlockSpec((1, TN, C_OUT), lambda b, t: (b, t, 0)),
        out_shape=jax.ShapeDtypeStruct((B, N, C_OUT), jnp.bfloat16),
    )(input, points, w1, w2)

    srows = pl.pallas_call(
        _s_body,
        grid=(B,),
        in_specs=[
            pl.BlockSpec((1, 3, NS), lambda b: (b, 0, 0)),
            pl.BlockSpec((3, C_OUT), lambda b: (0, 0)),
            pl.BlockSpec((1, C_OUT), lambda b: (0, 0)),
        ],
        out_specs=pl.BlockSpec((1, NS, C_OUT), lambda b: (b, 0, 0)),
        out_shape=jax.ShapeDtypeStruct((B, NS, C_OUT), jnp.bfloat16),
    )(support_points, w2, bbias.reshape(1, C_OUT))

    zflat = z.reshape(B * N, C_OUT)
    sflat = srows.reshape(B * NS, C_OUT)
    idx_flat = indices.astype(jnp.int32).reshape(-1)

    mesh = plsc.VectorSubcoreMesh(core_axis_name="c", subcore_axis_name="s",
                                  num_cores=NC, num_subcores=NSUB)
    out = pl.kernel(
        _sc_body,
        out_type=jax.ShapeDtypeStruct((B, C_OUT, NS), jnp.float32),
        mesh=mesh,
        compiler_params=pltpu.CompilerParams(use_tc_tiling_on_sc=False,
                                             needs_layout_passes=False),
        scratch_types=[
            pltpu.VMEM((SUBC * K,), jnp.int32),
            pltpu.VMEM((SUBC, C_OUT), jnp.bfloat16),
            pltpu.VMEM((GRP * K, C_OUT), jnp.bfloat16),
            pltpu.VMEM((GRP * K, C_OUT), jnp.bfloat16),
            pltpu.VMEM((C_OUT, SUBC), jnp.float32),
            pltpu.SemaphoreType.DMA,
            pltpu.SemaphoreType.DMA,
        ],
    )(zflat, idx_flat, sflat)

    return (out, support_points, indices)
